# baseline (speedup 1.0000x reference)
import functools

import jax
import jax.numpy as jnp
from jax import lax
from jax.experimental import pallas as pl
from jax.experimental.pallas import tpu as pltpu
from jax.experimental.pallas import tpu_sc as plsc

NC = 2
NS = 16
NW = NC * NS
CHUNK = 80


def _sc_mesh():
    return plsc.VectorSubcoreMesh(core_axis_name="c", subcore_axis_name="s")


def _sub_slices(n):
    rpa = ((n // NS) // 8) * 8
    tail = n - NS * rpa
    assert tail % 8 == 0 and tail >= 0
    return rpa, tail


def _copy_striped(n, sid, copy_fn):
    rpa, tail = _sub_slices(n)
    copy_fn(pl.multiple_of(sid * rpa, 8), rpa)
    if tail:
        @pl.when(sid == 0)
        def _():
            copy_fn(NS * rpa, tail)


def _make_deg_kernel(n, d, ept, nchunk):
    @functools.partial(
        pl.kernel,
        out_type=jax.ShapeDtypeStruct((NC, n, d), jnp.float32),
        mesh=_sc_mesh(),
        scratch_types=[
            pltpu.VMEM((nchunk, CHUNK), jnp.int32),
            pltpu.VMEM((CHUNK, d), jnp.float32),
            pltpu.VMEM_SHARED((n + 1, d), jnp.float32),
            pltpu.SemaphoreType.DMA,
        ],
    )
    def deg_kernel(u3, znd, ones_h, degp, uloc, ones_v, acc, dsem):
        cid = lax.axis_index("c")
        sid = lax.axis_index("s")
        gidx = cid * NS + sid
        _copy_striped(n, sid, lambda o, r: pltpu.sync_copy(
            znd.at[pl.ds(o, r)], acc.at[pl.ds(o, r)]))
        pltpu.sync_copy(ones_h, ones_v)
        pltpu.sync_copy(u3.at[gidx], uloc)
        plsc.subcore_barrier()
        dring = min(6, nchunk)

        def body(j, carry):
            pltpu.async_copy(ones_v, acc.at[uloc.at[j]], dsem, add=True)

            @pl.when(j >= dring)
            def _():
                pltpu.make_async_copy(ones_v, acc.at[uloc.at[0]], dsem).wait()

            return carry

        lax.fori_loop(0, nchunk, body, None)
        for _ in range(dring):
            pltpu.make_async_copy(ones_v, acc.at[uloc.at[0]], dsem).wait()
        plsc.subcore_barrier()
        _copy_striped(n, sid, lambda o, r: pltpu.sync_copy(
            acc.at[pl.ds(o, r)], degp.at[cid, pl.ds(o, r)]))

    return deg_kernel


CG = 8
RING = 3
SPC = CHUNK // CG


def _make_prop_kernel(n, d, ept, nchunk):
    assert CHUNK % CG == 0 and nchunk >= RING

    @functools.partial(
        pl.kernel,
        out_type=jax.ShapeDtypeStruct((NC, n, d), jnp.float32),
        mesh=_sc_mesh(),
        scratch_types=[
            pltpu.VMEM((ept,), jnp.int32),
            pltpu.VMEM((RING, CHUNK), jnp.int32),
            pltpu.VMEM((RING * CHUNK, d), jnp.float32),
            pltpu.VMEM_SHARED((n + 1, d), jnp.float32),
            pltpu.SemaphoreType.DMA,
            pltpu.SemaphoreType.DMA,
            pltpu.SemaphoreType.DMA,
        ],
    )
    def prop_kernel(y, u1, v3, znd, part, uloc, vidx, rows, acc, gsem, ssem, vsem):
        cid = lax.axis_index("c")
        sid = lax.axis_index("s")
        gidx = cid * NS + sid
        _copy_striped(n, sid, lambda o, r: pltpu.sync_copy(
            znd.at[pl.ds(o, r)], acc.at[pl.ds(o, r)]))
        pltpu.sync_copy(u1.at[pl.ds(pl.multiple_of(gidx * ept, 8), ept)], uloc)

        def chunk_rows(j):
            return rows.at[pl.ds(pl.multiple_of((j % RING) * CHUNK, 8), CHUNK)]

        def start_gathers(j):
            for k in range(SPC):
                idx = uloc.at[pl.ds(pl.multiple_of(j * CHUNK + k * CG, 8), CG)]
                dst = rows.at[pl.ds(
                    pl.multiple_of((j % RING) * CHUNK + k * CG, 8), CG)]
                pltpu.async_copy(y.at[idx], dst, gsem)

        def wait_gathers(j):
            for k in range(SPC):
                pltpu.make_async_copy(
                    y.at[pl.ds(0, CG)], rows.at[pl.ds(0, CG)], gsem).wait()

        def start_vload(j):
            pltpu.async_copy(v3.at[gidx, j], vidx.at[j % RING], vsem)

        def wait_vload(j):
            pltpu.make_async_copy(
                v3.at[gidx, 0], vidx.at[j % RING], vsem).wait()

        def start_scatter(j):
            pltpu.async_copy(chunk_rows(j), acc.at[vidx.at[j % RING]],
                             ssem, add=True)

        def drain_scatter():
            pltpu.make_async_copy(chunk_rows(0), acc.at[vidx.at[0]],
                                  ssem).wait()

        plsc.subcore_barrier()
        for jj in range(RING - 1):
            start_vload(jj)
            start_gathers(jj)

        def body(j, carry):
            wait_gathers(j)
            wait_vload(j)
            start_scatter(j)

            @pl.when(j + RING - 1 < nchunk)
            def _():
                @pl.when(j >= 1)
                def _():
                    drain_scatter()
                start_vload(j + RING - 1)
                start_gathers(j + RING - 1)

            return carry

        lax.fori_loop(0, nchunk, body, None)
        for _ in range(min(RING, nchunk)):
            drain_scatter()
        plsc.subcore_barrier()
        _copy_striped(n, sid, lambda o, r: pltpu.sync_copy(
            acc.at[pl.ds(o, r)], part.at[cid, pl.ds(o, r)]))

    return prop_kernel


def _dinv_from_degp(degp_blk):
    deg = degp_blk[0, 0] + 1.0
    return lax.rsqrt(deg)


def _tc_first_body(x_ref, degp_ref, w_ref, b_ref, y_ref):
    dinv = _dinv_from_degp(degp_ref[...])[:, None]
    xw = lax.dot_general(x_ref[...], w_ref[...], (((1,), (1,)), ((), ())),
                         preferred_element_type=jnp.float32)
    y_ref[...] = (xw + b_ref[...]) * dinv


def _tc_mid_body(p_ref, yprev_ref, degp_ref, w_ref, b_ref, y_ref):
    dinv = _dinv_from_degp(degp_ref[...])[:, None]
    p = p_ref[...]
    x = jax.nn.sigmoid((p[0] + p[1] + yprev_ref[...]) * dinv)
    xw = lax.dot_general(x, w_ref[...], (((1,), (1,)), ((), ())),
                         preferred_element_type=jnp.float32)
    y_ref[...] = (xw + b_ref[...]) * dinv


def _tc_final_body(p_ref, ylast_ref, degp_ref, out_ref):
    dinv = _dinv_from_degp(degp_ref[...])[:, None]
    p = p_ref[...]
    out_ref[...] = (p[0] + p[1] + ylast_ref[...]) * dinv


def kernel(vertices, edges, W0, b0, W1, b1, W2, b2):
    n, d = vertices.shape
    e = edges.shape[0]
    u = edges[:, 0].astype(jnp.int32)
    v = edges[:, 1].astype(jnp.int32)

    nchunk = -(-e // (NW * CHUNK))
    epad = NW * CHUNK * nchunk
    if epad != e:
        u_gather = jnp.concatenate([u, jnp.zeros((epad - e,), jnp.int32)])
        u_scatter = jnp.concatenate([u, jnp.full((epad - e,), n, jnp.int32)])
        v = jnp.concatenate([v, jnp.full((epad - e,), n, jnp.int32)])
    else:
        u_gather = u_scatter = u
    ept = nchunk * CHUNK
    u_deg3 = u_scatter.reshape(NW, nchunk, CHUNK)
    v3 = v.reshape(NW, nchunk, CHUNK)

    znd = jnp.zeros((n, d), jnp.float32)
    ones_h = jnp.ones((CHUNK, d), jnp.float32)

    bn = 1000 if n % 1000 == 0 else n
    cnt = _make_deg_kernel(n, d, ept, nchunk)(u_deg3, znd, ones_h)
    degv = (cnt[0, :, 0] + cnt[1, :, 0]).reshape(n // bn, 1, bn)
    prop = _make_prop_kernel(n, d, ept, nchunk)

    grid = (n // bn,)
    nd_spec = pl.BlockSpec((bn, d), lambda i: (i, 0))
    part_spec = pl.BlockSpec((NC, bn, d), lambda i: (0, i, 0))
    degp_spec = pl.BlockSpec((1, 1, bn), lambda i: (i, 0, 0))
    w_spec = pl.BlockSpec((d, d), lambda i: (0, 0))
    b_spec = pl.BlockSpec((1, d), lambda i: (0, 0))
    out_nd = jax.ShapeDtypeStruct((n, d), jnp.float32)

    tc_first = pl.pallas_call(
        _tc_first_body, grid=grid,
        in_specs=[nd_spec, degp_spec, w_spec, b_spec],
        out_specs=nd_spec, out_shape=out_nd)
    tc_mid = pl.pallas_call(
        _tc_mid_body, grid=grid,
        in_specs=[part_spec, nd_spec, degp_spec, w_spec, b_spec],
        out_specs=nd_spec, out_shape=out_nd)
    tc_final = pl.pallas_call(
        _tc_final_body, grid=grid,
        in_specs=[part_spec, nd_spec, degp_spec],
        out_specs=nd_spec, out_shape=out_nd)

    y = tc_first(vertices, degv, W0, b0.reshape(1, d))
    p = prop(y, u_gather, v3, znd)
    y = tc_mid(p, y, degv, W1, b1.reshape(1, d))
    p = prop(y, u_gather, v3, znd)
    y = tc_mid(p, y, degv, W2, b2.reshape(1, d))
    p = prop(y, u_gather, v3, znd)
    return tc_final(p, y, degv)


# speedup vs baseline: 1.1357x; 1.1357x over previous
import functools

import jax
import jax.numpy as jnp
from jax import lax
from jax.experimental import pallas as pl
from jax.experimental.pallas import tpu as pltpu
from jax.experimental.pallas import tpu_sc as plsc

NC = 2
NS = 16
NW = NC * NS
CHUNK = 80


def _sc_mesh():
    return plsc.VectorSubcoreMesh(core_axis_name="c", subcore_axis_name="s")


def _sub_slices(n):
    rpa = ((n // NS) // 8) * 8
    tail = n - NS * rpa
    assert tail % 8 == 0 and tail >= 0
    return rpa, tail


def _copy_striped(n, sid, copy_fn):
    rpa, tail = _sub_slices(n)
    copy_fn(pl.multiple_of(sid * rpa, 8), rpa)
    if tail:
        @pl.when(sid == 0)
        def _():
            copy_fn(NS * rpa, tail)


def _lane_shift_gather(x, lanes, delta):
    idx = jnp.clip(lanes + delta, 0, 15)
    dn = lax.GatherDimensionNumbers(
        offset_dims=(), collapsed_slice_dims=(0,), start_index_map=(0,))
    return lax.gather(x, idx[:, None], dn, (1,),
                      mode=lax.GatherScatterMode.PROMISE_IN_BOUNDS)


def _make_deg_kernel(n, ept):
    assert n % 16 == 0 and ept % 16 == 0

    @functools.partial(
        pl.kernel,
        out_type=jax.ShapeDtypeStruct((NW * n,), jnp.float32),
        mesh=_sc_mesh(),
        scratch_types=[
            pltpu.VMEM((ept,), jnp.int32),
            pltpu.VMEM((n + 16,), jnp.float32),
        ],
        compiler_params=pltpu.CompilerParams(needs_layout_passes=False),
    )
    def deg_kernel(u1, degf, uloc, degl):
        cid = lax.axis_index("c")
        sid = lax.axis_index("s")
        gidx = cid * NS + sid
        zeros16 = jnp.zeros((16,), jnp.float32)
        lanes = lax.iota(jnp.int32, 16)

        def zero(i, carry):
            degl[pl.ds(i * 16, 16)] = zeros16
            return carry

        lax.fori_loop(0, (n + 16) // 16, zero, None)
        pltpu.sync_copy(u1.at[pl.ds(pl.multiple_of(gidx * ept, 8), ept)], uloc)

        def body(k, carry):
            idx = uloc[pl.ds(pl.multiple_of(k * 16, 8), 16)]
            sk, _ = plsc.sort_key_val(idx, idx)
            prev = _lane_shift_gather(sk, lanes, -1)
            nxt = _lane_shift_gather(sk, lanes, 1)
            start = (lanes == 0) | (prev != sk)
            end = (lanes == 15) | (nxt != sk)
            start_lane = plsc.cummax(jnp.where(start, lanes, 0))
            cnt = (lanes - start_lane + 1).astype(jnp.float32)
            old = plsc.load_gather(degl, [sk], mask=end)
            plsc.store_scatter(degl, [sk], old + cnt, mask=end)
            return carry

        lax.fori_loop(0, ept // 16, body, None)
        pltpu.sync_copy(degl.at[pl.ds(0, n)],
                        degf.at[pl.ds(pl.multiple_of(gidx * n, 8), n)])

    return deg_kernel


CG = 16
RING = 3
SPC = CHUNK // CG


def _make_prop_kernel(n, d, ept, nchunk):
    assert CHUNK % CG == 0 and nchunk >= RING

    @functools.partial(
        pl.kernel,
        out_type=jax.ShapeDtypeStruct((NC, n, d), jnp.float32),
        mesh=_sc_mesh(),
        scratch_types=[
            pltpu.VMEM((ept,), jnp.int32),
            pltpu.VMEM((RING, CHUNK), jnp.int32),
            pltpu.VMEM((RING * CHUNK, d), jnp.float32),
            pltpu.VMEM_SHARED((n + 1, d), jnp.float32),
            pltpu.SemaphoreType.DMA,
            pltpu.SemaphoreType.DMA,
            pltpu.SemaphoreType.DMA,
        ],
    )
    def prop_kernel(y, u1, v3, znd, part, uloc, vidx, rows, acc, gsem, ssem, vsem):
        cid = lax.axis_index("c")
        sid = lax.axis_index("s")
        gidx = cid * NS + sid
        _copy_striped(n, sid, lambda o, r: pltpu.sync_copy(
            znd.at[pl.ds(o, r)], acc.at[pl.ds(o, r)]))
        pltpu.sync_copy(u1.at[pl.ds(pl.multiple_of(gidx * ept, 8), ept)], uloc)

        def chunk_rows(j):
            return rows.at[pl.ds(pl.multiple_of((j % RING) * CHUNK, 8), CHUNK)]

        def start_gathers(j):
            for k in range(SPC):
                idx = uloc.at[pl.ds(pl.multiple_of(j * CHUNK + k * CG, 8), CG)]
                dst = rows.at[pl.ds(
                    pl.multiple_of((j % RING) * CHUNK + k * CG, 8), CG)]
                pltpu.async_copy(y.at[idx], dst, gsem)

        def wait_gathers(j):
            for k in range(SPC):
                pltpu.make_async_copy(
                    y.at[pl.ds(0, CG)], rows.at[pl.ds(0, CG)], gsem).wait()

        def start_vload(j):
            pltpu.async_copy(v3.at[gidx, j], vidx.at[j % RING], vsem)

        def wait_vload(j):
            pltpu.make_async_copy(
                v3.at[gidx, 0], vidx.at[j % RING], vsem).wait()

        def start_scatter(j):
            pltpu.async_copy(chunk_rows(j), acc.at[vidx.at[j % RING]],
                             ssem, add=True)

        def drain_scatter():
            pltpu.make_async_copy(chunk_rows(0), acc.at[vidx.at[0]],
                                  ssem).wait()

        plsc.subcore_barrier()
        for jj in range(RING - 1):
            start_vload(jj)
            start_gathers(jj)

        def body(j, carry):
            wait_gathers(j)
            wait_vload(j)
            start_scatter(j)

            @pl.when(j + RING - 1 < nchunk)
            def _():
                @pl.when(j >= 1)
                def _():
                    drain_scatter()
                start_vload(j + RING - 1)
                start_gathers(j + RING - 1)

            return carry

        lax.fori_loop(0, nchunk, body, None)
        for _ in range(min(RING, nchunk)):
            drain_scatter()
        plsc.subcore_barrier()
        _copy_striped(n, sid, lambda o, r: pltpu.sync_copy(
            acc.at[pl.ds(o, r)], part.at[cid, pl.ds(o, r)]))

    return prop_kernel


def _dinv_from_degp(degp_blk):
    deg = jnp.sum(degp_blk[0], axis=0) + 1.0
    return lax.rsqrt(deg)


def _tc_first_body(x_ref, degp_ref, w_ref, b_ref, y_ref):
    dinv = _dinv_from_degp(degp_ref[...])[:, None]
    xw = lax.dot_general(x_ref[...], w_ref[...], (((1,), (1,)), ((), ())),
                         preferred_element_type=jnp.float32)
    y_ref[...] = (xw + b_ref[...]) * dinv


def _tc_mid_body(p_ref, yprev_ref, degp_ref, w_ref, b_ref, y_ref):
    dinv = _dinv_from_degp(degp_ref[...])[:, None]
    p = p_ref[...]
    x = jax.nn.sigmoid((p[0] + p[1] + yprev_ref[...]) * dinv)
    xw = lax.dot_general(x, w_ref[...], (((1,), (1,)), ((), ())),
                         preferred_element_type=jnp.float32)
    y_ref[...] = (xw + b_ref[...]) * dinv


def _tc_final_body(p_ref, ylast_ref, degp_ref, out_ref):
    dinv = _dinv_from_degp(degp_ref[...])[:, None]
    p = p_ref[...]
    out_ref[...] = (p[0] + p[1] + ylast_ref[...]) * dinv


def kernel(vertices, edges, W0, b0, W1, b1, W2, b2):
    n, d = vertices.shape
    e = edges.shape[0]
    u = edges[:, 0].astype(jnp.int32)
    v = edges[:, 1].astype(jnp.int32)

    nchunk = -(-e // (NW * CHUNK))
    epad = NW * CHUNK * nchunk
    if epad != e:
        u_gather = jnp.concatenate([u, jnp.zeros((epad - e,), jnp.int32)])
        u_scatter = jnp.concatenate([u, jnp.full((epad - e,), n, jnp.int32)])
        v = jnp.concatenate([v, jnp.full((epad - e,), n, jnp.int32)])
    else:
        u_gather = u_scatter = u
    ept = nchunk * CHUNK
    v3 = v.reshape(NW, nchunk, CHUNK)

    znd = jnp.zeros((n, d), jnp.float32)

    bn = 1000 if n % 1000 == 0 else n
    cnt = _make_deg_kernel(n, ept)(u_scatter)
    degv = cnt.reshape(NW, n // bn, bn).transpose(1, 0, 2)
    prop = _make_prop_kernel(n, d, ept, nchunk)

    grid = (n // bn,)
    nd_spec = pl.BlockSpec((bn, d), lambda i: (i, 0))
    part_spec = pl.BlockSpec((NC, bn, d), lambda i: (0, i, 0))
    degp_spec = pl.BlockSpec((1, NW, bn), lambda i: (i, 0, 0))
    w_spec = pl.BlockSpec((d, d), lambda i: (0, 0))
    b_spec = pl.BlockSpec((1, d), lambda i: (0, 0))
    out_nd = jax.ShapeDtypeStruct((n, d), jnp.float32)

    tc_first = pl.pallas_call(
        _tc_first_body, grid=grid,
        in_specs=[nd_spec, degp_spec, w_spec, b_spec],
        out_specs=nd_spec, out_shape=out_nd)
    tc_mid = pl.pallas_call(
        _tc_mid_body, grid=grid,
        in_specs=[part_spec, nd_spec, degp_spec, w_spec, b_spec],
        out_specs=nd_spec, out_shape=out_nd)
    tc_final = pl.pallas_call(
        _tc_final_body, grid=grid,
        in_specs=[part_spec, nd_spec, degp_spec],
        out_specs=nd_spec, out_shape=out_nd)

    y = tc_first(vertices, degv, W0, b0.reshape(1, d))
    p = prop(y, u_gather, v3, znd)
    y = tc_mid(p, y, degv, W1, b1.reshape(1, d))
    p = prop(y, u_gather, v3, znd)
    y = tc_mid(p, y, degv, W2, b2.reshape(1, d))
    p = prop(y, u_gather, v3, znd)
    return tc_final(p, y, degv)
